# baseline (speedup 1.0000x reference)
import functools

import jax
import jax.numpy as jnp
from jax import lax
from jax.experimental import pallas as pl
from jax.experimental.pallas import tpu as pltpu
from jax.experimental.pallas import tpu_sc as plsc

NUM_WORKERS = 32
CHUNK_V = 16384
PACK = 4


def _transpose_body(x_ref, y_ref):
    x = x_ref[0]
    q = CHUNK_V // PACK
    z = jnp.concatenate([x[:, q * a:q * (a + 1)] for a in range(PACK)], axis=0)
    y_ref[0] = jnp.swapaxes(z, 0, 1)


def _tc_pack_tables(tT, f0, nf):
    _, D, V = tT.shape
    nc = (V + CHUNK_V - 1) // CHUNK_V
    rpf = nc * (CHUNK_V // PACK)
    return pl.pallas_call(
        _transpose_body,
        grid=(nf, nc),
        in_specs=[pl.BlockSpec((1, D, CHUNK_V), lambda f, c: (f + f0, 0, c))],
        out_specs=pl.BlockSpec((1, CHUNK_V // PACK, PACK * D),
                               lambda f, c: (f, c, 0)),
        out_shape=jax.ShapeDtypeStruct((nf, rpf, PACK * D), jnp.float32),
    )(tT)


def _sc_gather(flat_tables, idx, chunk):
    total = idx.shape[0]
    D = flat_tables.shape[1]
    per_w = total // NUM_WORKERS
    n_chunks = per_w // chunk
    mesh = plsc.VectorSubcoreMesh(core_axis_name="c", subcore_axis_name="s")

    @functools.partial(
        pl.kernel,
        mesh=mesh,
        out_type=jax.ShapeDtypeStruct((total, D), jnp.float32),
        compiler_params=pltpu.CompilerParams(use_tc_tiling_on_sc=False),
        scratch_types=[
            pltpu.VMEM((per_w,), jnp.int32),
            pltpu.VMEM((chunk, D), jnp.float32),
            pltpu.VMEM((chunk, D), jnp.float32),
            pltpu.SemaphoreType.DMA,
            pltpu.SemaphoreType.DMA,
        ],
    )
    def gather_kernel(tab_hbm, idx_hbm, out_hbm, idx_v, rows0, rows1, sem0, sem1):
        wid = lax.axis_index("s") * 2 + lax.axis_index("c")
        base = wid * per_w
        pltpu.sync_copy(idx_hbm.at[pl.ds(base, per_w)], idx_v)

        bufs, sems = (rows0, rows1), (sem0, sem1)
        handles = [None] * n_chunks
        handles[0] = pltpu.async_copy(
            tab_hbm.at[idx_v.at[pl.ds(0, chunk)]], bufs[0], sems[0])
        for c in range(n_chunks):
            handles[c].wait()
            if c + 1 < n_chunks:
                handles[c + 1] = pltpu.async_copy(
                    tab_hbm.at[idx_v.at[pl.ds((c + 1) * chunk, chunk)]],
                    bufs[(c + 1) % 2], sems[(c + 1) % 2])
            pltpu.sync_copy(bufs[c % 2],
                            out_hbm.at[pl.ds(base + c * chunk, chunk)])

    return gather_kernel(flat_tables, idx)


def _mlp_body(xg0_ref, xg1_ref, num_ref, w1a_ref, w1b_ref, w1n_ref, b1_ref,
              w2_ref, b2_ref, w3_ref, b3_ref, w4_ref, b4_ref, out_ref):
    h = jnp.dot(xg0_ref[...], w1a_ref[...], preferred_element_type=jnp.float32)
    h += jnp.dot(xg1_ref[...], w1b_ref[...], preferred_element_type=jnp.float32)
    h += jnp.dot(num_ref[...], w1n_ref[...], preferred_element_type=jnp.float32)
    h = jnp.maximum(h + b1_ref[...], 0.0)
    h = jnp.maximum(
        jnp.dot(h, w2_ref[...], preferred_element_type=jnp.float32) + b2_ref[...], 0.0)
    h = jnp.maximum(
        jnp.dot(h, w3_ref[...], preferred_element_type=jnp.float32) + b3_ref[...], 0.0)
    out_ref[...] = jnp.dot(h, w4_ref[...], preferred_element_type=jnp.float32) + b4_ref[...]


def _tc_mlp(xg0, xg1, num, w1a, w1b, w1n, b1, w2, b2, w3, b3, w4, b4, bm):
    B = xg0.shape[0]
    grid = (B // bm,)
    full = lambda a: pl.BlockSpec(a.shape, lambda i: (0,) * a.ndim)
    row = lambda a: pl.BlockSpec((bm, a.shape[1]), lambda i: (i, 0))
    out = pl.pallas_call(
        _mlp_body,
        grid=grid,
        in_specs=[
            row(xg0), row(xg1), row(num),
            full(w1a), full(w1b), full(w1n), full(b1), full(w2), full(b2),
            full(w3), full(b3), full(w4), full(b4),
        ],
        out_specs=pl.BlockSpec((bm, 1), lambda i: (i, 0)),
        out_shape=jax.ShapeDtypeStruct((B, 1), jnp.float32),
    )(xg0, xg1, num, w1a, w1b, w1n, b1, w2, b2, w3, b3, w4, b4)
    return out[:, 0]


def _packed_idx(cat_slice, rpf):
    nf = cat_slice.shape[1]
    q = CHUNK_V // PACK
    f_base = (jnp.arange(nf, dtype=jnp.int32) * (rpf * PACK))[None, :]
    v = cat_slice
    return (f_base + (v // CHUNK_V) * CHUNK_V + (v % q) * PACK
            + (v % CHUNK_V) // q).reshape(-1)


def kernel(cat_data, num_data, tables, W1, b1, W2, b2, W3, b3, W4, b4):
    B, NF = cat_data.shape
    V, D = tables.shape[1], tables.shape[2]
    nf0 = NF // 2
    nf1 = NF - nf0
    tT = jnp.swapaxes(tables, 1, 2)

    packed0 = _tc_pack_tables(tT, 0, nf0)
    packed1 = _tc_pack_tables(tT, nf0, nf1)
    rpf = packed0.shape[1]
    flat0 = packed0.reshape(nf0 * rpf * PACK, D)
    flat1 = packed1.reshape(nf1 * rpf * PACK, D)

    idx0 = _packed_idx(cat_data[:, :nf0], rpf)
    idx1 = _packed_idx(cat_data[:, nf0:], rpf)

    g0 = _sc_gather(flat0, idx0, chunk=832)
    g1 = _sc_gather(flat1, idx1, chunk=832)
    xg0 = g0.reshape(B, nf0 * D)
    xg1 = g1.reshape(B, nf1 * D)

    e0 = nf0 * D
    e1 = NF * D
    return _tc_mlp(
        xg0, xg1, num_data,
        W1[:e0], W1[e0:e1], W1[e1:],
        b1.reshape(1, -1), W2, b2.reshape(1, -1),
        W3, b3.reshape(1, -1), W4, b4.reshape(1, -1),
        bm=1024,
    )


# speedup vs baseline: 1.0348x; 1.0348x over previous
import functools

import jax
import jax.numpy as jnp
from jax import lax
from jax.experimental import pallas as pl
from jax.experimental.pallas import tpu as pltpu
from jax.experimental.pallas import tpu_sc as plsc

NUM_WORKERS = 32
CHUNK_V = 16384
PACK = 4


def _transpose_body(x_ref, y_ref):
    x = x_ref[0]
    q = CHUNK_V // PACK
    z = jnp.concatenate([x[:, q * a:q * (a + 1)] for a in range(PACK)], axis=0)
    y_ref[0] = jnp.swapaxes(z, 0, 1)


def _tc_pack_tables(tT, f0, nf):
    _, D, V = tT.shape
    nc = (V + CHUNK_V - 1) // CHUNK_V
    rpf = nc * (CHUNK_V // PACK)
    return pl.pallas_call(
        _transpose_body,
        grid=(nf, nc),
        in_specs=[pl.BlockSpec((1, D, CHUNK_V), lambda f, c: (f + f0, 0, c))],
        out_specs=pl.BlockSpec((1, CHUNK_V // PACK, PACK * D),
                               lambda f, c: (f, c, 0)),
        out_shape=jax.ShapeDtypeStruct((nf, rpf, PACK * D), jnp.float32),
    )(tT)


def _sc_gather(flat_tables, idx, chunk):
    total = idx.shape[0]
    D = flat_tables.shape[1]
    per_w = total // NUM_WORKERS
    n_chunks = per_w // chunk
    mesh = plsc.VectorSubcoreMesh(core_axis_name="c", subcore_axis_name="s")

    @functools.partial(
        pl.kernel,
        mesh=mesh,
        out_type=jax.ShapeDtypeStruct((total, D), jnp.float32),
        compiler_params=pltpu.CompilerParams(use_tc_tiling_on_sc=False),
        scratch_types=[
            pltpu.VMEM((per_w,), jnp.int32),
            pltpu.VMEM((chunk, D), jnp.float32),
            pltpu.VMEM((chunk, D), jnp.float32),
            pltpu.SemaphoreType.DMA,
            pltpu.SemaphoreType.DMA,
        ],
    )
    def gather_kernel(tab_hbm, idx_hbm, out_hbm, idx_v, rows0, rows1, sem0, sem1):
        wid = lax.axis_index("s") * 2 + lax.axis_index("c")
        base = wid * per_w
        pltpu.sync_copy(idx_hbm.at[pl.ds(base, per_w)], idx_v)

        bufs, sems = (rows0, rows1), (sem0, sem1)
        handles = [None] * n_chunks
        handles[0] = pltpu.async_copy(
            tab_hbm.at[idx_v.at[pl.ds(0, chunk)]], bufs[0], sems[0])
        for c in range(n_chunks):
            handles[c].wait()
            if c + 1 < n_chunks:
                handles[c + 1] = pltpu.async_copy(
                    tab_hbm.at[idx_v.at[pl.ds((c + 1) * chunk, chunk)]],
                    bufs[(c + 1) % 2], sems[(c + 1) % 2])
            pltpu.sync_copy(bufs[c % 2],
                            out_hbm.at[pl.ds(base + c * chunk, chunk)])

    return gather_kernel(flat_tables, idx)


def _mlp_body(xg_ref, num_ref, w1e_ref, w1n_ref, b1_ref,
              w2_ref, b2_ref, w3_ref, b3_ref, w4_ref, b4_ref, out_ref):
    h = jnp.dot(xg_ref[...], w1e_ref[...], preferred_element_type=jnp.float32)
    h += jnp.dot(num_ref[...], w1n_ref[...], preferred_element_type=jnp.float32)
    h = jnp.maximum(h + b1_ref[...], 0.0)
    h = jnp.maximum(
        jnp.dot(h, w2_ref[...], preferred_element_type=jnp.float32) + b2_ref[...], 0.0)
    h = jnp.maximum(
        jnp.dot(h, w3_ref[...], preferred_element_type=jnp.float32) + b3_ref[...], 0.0)
    out_ref[...] = jnp.dot(h, w4_ref[...], preferred_element_type=jnp.float32) + b4_ref[...]


def _tc_mlp(xg, num, w1e, w1n, b1, w2, b2, w3, b3, w4, b4, bm):
    B = xg.shape[0]
    grid = (B // bm,)
    full = lambda a: pl.BlockSpec(a.shape, lambda i: (0,) * a.ndim)
    row = lambda a: pl.BlockSpec((bm, a.shape[1]), lambda i: (i, 0))
    out = pl.pallas_call(
        _mlp_body,
        grid=grid,
        in_specs=[
            row(xg), row(num),
            full(w1e), full(w1n), full(b1), full(w2), full(b2),
            full(w3), full(b3), full(w4), full(b4),
        ],
        out_specs=pl.BlockSpec((bm, 1), lambda i: (i, 0)),
        out_shape=jax.ShapeDtypeStruct((B, 1), jnp.float32),
    )(xg, num, w1e, w1n, b1, w2, b2, w3, b3, w4, b4)
    return out[:, 0]


def _packed_idx(cat_slice, rpf):
    nf = cat_slice.shape[1]
    q = CHUNK_V // PACK
    f_base = (jnp.arange(nf, dtype=jnp.int32) * (rpf * PACK))[None, :]
    v = cat_slice
    return (f_base + (v // CHUNK_V) * CHUNK_V + (v % q) * PACK
            + (v % CHUNK_V) // q).reshape(-1)


def kernel(cat_data, num_data, tables, W1, b1, W2, b2, W3, b3, W4, b4):
    B, NF = cat_data.shape
    V, D = tables.shape[1], tables.shape[2]
    tT = jnp.swapaxes(tables, 1, 2)

    packed = _tc_pack_tables(tT, 0, NF)
    rpf = packed.shape[1]
    flat = packed.reshape(NF * rpf * PACK, D)

    idx = _packed_idx(cat_data, rpf)

    g = _sc_gather(flat, idx, chunk=1664)
    xg = g.reshape(B, NF * D)

    e1 = NF * D
    return _tc_mlp(
        xg, num_data,
        W1[:e1], W1[e1:],
        b1.reshape(1, -1), W2, b2.reshape(1, -1),
        W3, b3.reshape(1, -1), W4, b4.reshape(1, -1),
        bm=1024,
    )


# speedup vs baseline: 1.0453x; 1.0101x over previous
import functools

import jax
import jax.numpy as jnp
from jax import lax
from jax.experimental import pallas as pl
from jax.experimental.pallas import tpu as pltpu
from jax.experimental.pallas import tpu_sc as plsc

NUM_WORKERS = 32
CHUNK_V = 16384
PACK = 4


def _transpose_body(x_ref, y_ref):
    x = x_ref[0]
    q = CHUNK_V // PACK
    z = jnp.concatenate([x[:, q * a:q * (a + 1)] for a in range(PACK)], axis=0)
    y_ref[0] = jnp.swapaxes(z, 0, 1)


def _tc_pack_tables(tT, f0, nf):
    _, D, V = tT.shape
    nc = (V + CHUNK_V - 1) // CHUNK_V
    rpf = nc * (CHUNK_V // PACK)
    return pl.pallas_call(
        _transpose_body,
        grid=(nf, nc),
        in_specs=[pl.BlockSpec((1, D, CHUNK_V), lambda f, c: (f + f0, 0, c))],
        out_specs=pl.BlockSpec((1, CHUNK_V // PACK, PACK * D),
                               lambda f, c: (f, c, 0)),
        out_shape=jax.ShapeDtypeStruct((nf, rpf, PACK * D), jnp.float32),
    )(tT)


def _sc_gather(flat_tables, idx, chunk):
    total = idx.shape[0]
    D = flat_tables.shape[1]
    per_w = total // NUM_WORKERS
    n_chunks = per_w // chunk
    mesh = plsc.VectorSubcoreMesh(core_axis_name="c", subcore_axis_name="s")

    @functools.partial(
        pl.kernel,
        mesh=mesh,
        out_type=jax.ShapeDtypeStruct((total, D), jnp.float32),
        compiler_params=pltpu.CompilerParams(use_tc_tiling_on_sc=False),
        scratch_types=[
            pltpu.VMEM((per_w,), jnp.int32),
            pltpu.VMEM((chunk, D), jnp.float32),
            pltpu.VMEM((chunk, D), jnp.float32),
            pltpu.SemaphoreType.DMA,
            pltpu.SemaphoreType.DMA,
        ],
    )
    def gather_kernel(tab_hbm, idx_hbm, out_hbm, idx_v, rows0, rows1, sem0, sem1):
        wid = lax.axis_index("s") * 2 + lax.axis_index("c")
        base = wid * per_w
        pltpu.sync_copy(idx_hbm.at[pl.ds(base, per_w)], idx_v)

        bufs, sems = (rows0, rows1), (sem0, sem1)
        handles = [None] * n_chunks
        handles[0] = pltpu.async_copy(
            tab_hbm.at[idx_v.at[pl.ds(0, chunk)]], bufs[0], sems[0])
        for c in range(n_chunks):
            handles[c].wait()
            if c + 1 < n_chunks:
                handles[c + 1] = pltpu.async_copy(
                    tab_hbm.at[idx_v.at[pl.ds((c + 1) * chunk, chunk)]],
                    bufs[(c + 1) % 2], sems[(c + 1) % 2])
            pltpu.sync_copy(bufs[c % 2],
                            out_hbm.at[pl.ds(base + c * chunk, chunk)])

    return gather_kernel(flat_tables, idx)


def _mlp_body(xg_ref, num_ref, w1e_ref, w1n_ref, b1_ref,
              w2_ref, b2_ref, w3_ref, b3_ref, w4_ref, b4_ref, out_ref):
    h = jnp.dot(xg_ref[...], w1e_ref[...], preferred_element_type=jnp.float32)
    h += jnp.dot(num_ref[...], w1n_ref[...], preferred_element_type=jnp.float32)
    h = jnp.maximum(h + b1_ref[...], 0.0)
    h = jnp.maximum(
        jnp.dot(h, w2_ref[...], preferred_element_type=jnp.float32) + b2_ref[...], 0.0)
    h = jnp.maximum(
        jnp.dot(h, w3_ref[...], preferred_element_type=jnp.float32) + b3_ref[...], 0.0)
    out_ref[...] = jnp.dot(h, w4_ref[...], preferred_element_type=jnp.float32) + b4_ref[...]


def _tc_mlp(xg, num, w1e, w1n, b1, w2, b2, w3, b3, w4, b4, bm):
    B = xg.shape[0]
    grid = (B // bm,)
    full = lambda a: pl.BlockSpec(a.shape, lambda i: (0,) * a.ndim)
    row = lambda a: pl.BlockSpec((bm, a.shape[1]), lambda i: (i, 0))
    out = pl.pallas_call(
        _mlp_body,
        grid=grid,
        in_specs=[
            row(xg), row(num),
            full(w1e), full(w1n), full(b1), full(w2), full(b2),
            full(w3), full(b3), full(w4), full(b4),
        ],
        out_specs=pl.BlockSpec((bm, 1), lambda i: (i, 0)),
        out_shape=jax.ShapeDtypeStruct((B, 1), jnp.float32),
    )(xg, num, w1e, w1n, b1, w2, b2, w3, b3, w4, b4)
    return out[:, 0]


def _packed_idx(cat_slice, rpf):
    nf = cat_slice.shape[1]
    q = CHUNK_V // PACK
    f_base = (jnp.arange(nf, dtype=jnp.int32) * (rpf * PACK))[None, :]
    v = cat_slice
    return (f_base + (v // CHUNK_V) * CHUNK_V + (v % q) * PACK
            + (v % CHUNK_V) // q).reshape(-1)


def kernel(cat_data, num_data, tables, W1, b1, W2, b2, W3, b3, W4, b4):
    B, NF = cat_data.shape
    V, D = tables.shape[1], tables.shape[2]
    tT = jnp.swapaxes(tables, 1, 2)

    packed = _tc_pack_tables(tT, 0, NF)
    rpf = packed.shape[1]
    flat = packed.reshape(NF * rpf * PACK, D)

    idx = _packed_idx(cat_data, rpf)

    g = _sc_gather(flat, idx, chunk=1664)
    xg = g.reshape(B, NF * D)

    e1 = NF * D
    return _tc_mlp(
        xg, num_data,
        W1[:e1], W1[e1:],
        b1.reshape(1, -1), W2, b2.reshape(1, -1),
        W3, b3.reshape(1, -1), W4, b4.reshape(1, -1),
        bm=2048,
    )
